# baseline (speedup 1.0000x reference)
import functools
import jax
import jax.numpy as jnp
import numpy as np
from jax import lax
from jax.experimental import pallas as pl
from jax.experimental.pallas import tpu as pltpu
from jax.experimental.pallas import tpu_sc as plsc

_HIDDEN = 32
_FRAC = 0.35
_MIN_K = 6
_B = 64
_T = 8192
_D = 128
_CHUNK = 1024
_NCH = _T // _CHUNK

_INT_MIN = np.int32(-2147483648)
_KEY_NEG_INF = np.int32(-2139095041)

_GCH = 128


def _score_kernel(nb_ref, len_ref, x_ref, W1_ref, b1_ref, W2_ref, key_ref):
    b = pl.program_id(0)
    c = pl.program_id(1)
    l = len_ref[b]

    @pl.when(l > c * _CHUNK)
    def _():
        xc = x_ref[0]
        h = jnp.tanh(
            jnp.dot(xc, W1_ref[...], preferred_element_type=jnp.float32)
            + b1_ref[...]
        )
        sc = jnp.dot(h, W2_ref[...], preferred_element_type=jnp.float32)
        s2 = sc.reshape(8, 128)
        i = lax.bitcast_convert_type(s2, jnp.int32)
        skey = i ^ (lax.shift_right_arithmetic(i, 31) & jnp.int32(0x7FFFFFFF))
        key_ref[0, 0] = skey


def _radix_kernel(len_ref, key_ref, prm_ref, kf_ref):
    sk = key_ref[...].reshape(_B, _T // 128, 128)
    g = lax.broadcasted_iota(jnp.int32, (_B, _T // 128, 128), 1)
    lane = lax.broadcasted_iota(jnp.int32, (_B, _T // 128, 128), 2)
    t = g * 128 + lane
    lv = len_ref[...]
    lcol = lv.reshape(_B, 1, 1)
    sk = jnp.where(t < lcol, sk, jnp.int32(_KEY_NEG_INF))
    lf = lv.astype(jnp.float32) * jnp.float32(_FRAC)
    ki = lf.astype(jnp.int32)
    ki = ki + (ki.astype(jnp.float32) < lf).astype(jnp.int32)
    k = jnp.clip(ki, _MIN_K, _T)

    p = jnp.zeros((_B, 1), jnp.int32)
    for bit in range(31, -1, -1):
        cpat = p | (jnp.int32(1) << bit)
        cval = cpat ^ jnp.int32(_INT_MIN)
        cmp = (sk >= cval[:, :, None]).astype(jnp.int32)
        cnt = jnp.sum(jnp.sum(cmp, axis=1), axis=1, keepdims=True)
        p = jnp.where(cnt >= k, cpat, p)
    theta = p ^ jnp.int32(_INT_MIN)

    cgt = jnp.sum(
        jnp.sum((sk > theta[:, :, None]).astype(jnp.int32), axis=1),
        axis=1, keepdims=True,
    )
    m = k - cgt

    ones16 = jnp.ones((1, 16), jnp.int32)
    prm_ref[:, 0:16] = theta * ones16
    prm_ref[:, 16:32] = m * ones16
    prm_ref[:, 32:48] = lv * ones16
    kf_ref[...] = k.astype(jnp.float32) * jnp.ones((1, 16), jnp.float32)


def _sc_pool_kernel(key_hbm, prm_hbm, kf_hbm, xflat_hbm, out_hbm,
                    key_v, idx_v, rows_v, prm_v, kf_v, out_v, sem):
    wid = lax.axis_index("s") * 2 + lax.axis_index("c")

    for rb in range(2):
        b = wid + rb * 32
        pltpu.sync_copy(key_hbm.at[b], key_v)
        pltpu.sync_copy(prm_hbm.at[b], prm_v)
        pltpu.sync_copy(kf_hbm.at[b], kf_v)
        thv = prm_v[pl.ds(0, 16)]
        mv = prm_v[pl.ds(16, 16)]
        lv = prm_v[pl.ds(32, 16)]
        base = b * _T

        def filt(i, carry):
            off, eqc, last_idx = carry
            kv = key_v[pl.ds(i * 16, 16)]
            tvec = lax.iota(jnp.int32, 16) + i * 16
            kve = jnp.where(tvec < lv, kv, jnp.int32(_KEY_NEG_INF))
            gt = kve > thv
            eq = kve == thv
            eqi = eq.astype(jnp.int32)
            rank = plsc.cumsum(eqi) - eqi + eqc
            sel = gt | (eq & (rank < mv))
            gidx = tvec + base
            plsc.store_compressed(idx_v.at[pl.ds(off, 16)], gidx, mask=sel)
            nsel = jnp.sum(sel.astype(jnp.int32))
            neq = jnp.sum(eqi)
            li = jnp.max(jnp.where(sel, gidx, jnp.int32(-1)))
            return (off + nsel, eqc + neq, jnp.maximum(last_idx, li))

        off, _, last_idx = lax.fori_loop(
            0, _T // 16, filt, (jnp.int32(0), jnp.int32(0), jnp.int32(-1))
        )

        splat_last = jnp.full((16,), 1, jnp.int32) * last_idx
        for j in range(_GCH // 16):
            idx_v[pl.ds(off + j * 16, 16)] = splat_last

        nch = (off + _GCH - 1) // _GCH

        def gather_chunk(ci, accs):
            pltpu.async_copy(
                xflat_hbm.at[idx_v.at[pl.ds(ci * _GCH, _GCH)]], rows_v, sem
            ).wait()

            def addrow(r, accs2):
                row = tuple(
                    accs2[d] + rows_v[r, pl.ds(d * 16, 16)] for d in range(8)
                )
                return row

            return lax.fori_loop(0, _GCH, addrow, accs)

        acc0 = tuple(jnp.zeros((16,), jnp.float32) for _ in range(8))
        accs = lax.fori_loop(0, nch, gather_chunk, acc0)

        pad = (nch * _GCH - off).astype(jnp.float32)
        kf = kf_v[...]
        for d in range(8):
            corr = rows_v[_GCH - 1, pl.ds(d * 16, 16)] * pad
            out_v[pl.ds(d * 16, 16)] = (accs[d] - corr) / kf

        pltpu.sync_copy(out_v, out_hbm.at[b])


def _stages12(x, lengths, W1, b1, W2):
    B, T, D = x.shape
    b1r = b1.reshape(1, _HIDDEN).astype(jnp.float32)
    nb = jnp.clip((lengths + _CHUNK - 1) // _CHUNK, 1, _NCH)

    keys = pl.pallas_call(
        _score_kernel,
        grid_spec=pltpu.PrefetchScalarGridSpec(
            num_scalar_prefetch=1,
            grid=(B, _NCH),
            in_specs=[
                pl.BlockSpec(memory_space=pltpu.SMEM),
                pl.BlockSpec(
                    (1, _CHUNK, D),
                    lambda b, c, nb_ref: (b, jnp.minimum(c, nb_ref[b] - 1), 0),
                ),
                pl.BlockSpec((D, _HIDDEN), lambda b, c, nb_ref: (0, 0)),
                pl.BlockSpec((1, _HIDDEN), lambda b, c, nb_ref: (0, 0)),
                pl.BlockSpec((_HIDDEN, 1), lambda b, c, nb_ref: (0, 0)),
            ],
            out_specs=pl.BlockSpec(
                (1, 1, 8, 128), lambda b, c, nb_ref: (b, c, 0, 0)
            ),
        ),
        out_shape=jax.ShapeDtypeStruct((B, _NCH, 8, 128), jnp.int32),
    )(nb, lengths, x, W1, b1r, W2)

    prm, kf = pl.pallas_call(
        _radix_kernel,
        in_specs=[
            pl.BlockSpec((B, 1), lambda: (0, 0)),
            pl.BlockSpec((B, _NCH, 8, 128), lambda: (0, 0, 0, 0)),
        ],
        out_specs=[
            pl.BlockSpec((B, 48), lambda: (0, 0)),
            pl.BlockSpec((B, 16), lambda: (0, 0)),
        ],
        out_shape=[
            jax.ShapeDtypeStruct((B, 48), jnp.int32),
            jax.ShapeDtypeStruct((B, 16), jnp.float32),
        ],
    )(lengths.reshape(B, 1), keys)
    return keys, prm, kf


def kernel(x, lengths, W1, b1, W2):
    B, T, D = x.shape
    lengths = lengths.astype(jnp.int32)
    keys, prm, kf = _stages12(x, lengths, W1, b1, W2)

    keys_flat = keys.reshape(B, T)
    xflat = x.reshape(B * T, D)

    mesh = plsc.VectorSubcoreMesh(core_axis_name="c", subcore_axis_name="s")
    pooled = pl.kernel(
        _sc_pool_kernel,
        mesh=mesh,
        compiler_params=pltpu.CompilerParams(needs_layout_passes=False),
        out_type=jax.ShapeDtypeStruct((B, D), jnp.float32),
        scratch_types=[
            pltpu.VMEM((T,), jnp.int32),
            pltpu.VMEM((4096 + _GCH,), jnp.int32),
            pltpu.VMEM((_GCH, D), jnp.float32),
            pltpu.VMEM((48,), jnp.int32),
            pltpu.VMEM((16,), jnp.float32),
            pltpu.VMEM((D,), jnp.float32),
            pltpu.SemaphoreType.DMA,
        ],
    )(keys_flat, prm, kf, xflat)
    return pooled


# speedup vs baseline: 1.6777x; 1.6777x over previous
import jax
import jax.numpy as jnp
import numpy as np
from jax import lax
from jax.experimental import pallas as pl
from jax.experimental.pallas import tpu as pltpu

_HIDDEN = 32
_FRAC = 0.35
_MIN_K = 6
_RB = 64
_CB = 128
_CHUNK = 1024
_NCH = 8

_INT_MIN = np.int32(-2147483648)

_ROUNDS = [(29, 3), (26, 3), (23, 3), (20, 3), (17, 3), (14, 3), (11, 3),
           (8, 3), (5, 3), (2, 3), (0, 2)]


def _row_kernel(len_ref, xa_ref, xb_ref, W1_ref, b1_ref, W2_ref, out_ref,
                s_scr):
    b = pl.program_id(0)
    T = _RB * _CB
    l = len_ref[b]

    for c in range(_NCH):
        @pl.when(l > c * _CHUNK)
        def _():
            half = xa_ref if c < _NCH // 2 else xb_ref
            cc_ = c if c < _NCH // 2 else c - _NCH // 2
            xc = half[0][cc_ * _CHUNK:(cc_ + 1) * _CHUNK, :]
            h = jnp.tanh(
                jnp.dot(xc, W1_ref[...], preferred_element_type=jnp.float32)
                + b1_ref[...]
            )
            sc = jnp.dot(h, W2_ref[...], preferred_element_type=jnp.float32)
            s_scr[c * 8:(c + 1) * 8, :] = sc.reshape(8, _CB)

    r = lax.broadcasted_iota(jnp.int32, (_RB, _CB), 0)
    cc0 = lax.broadcasted_iota(jnp.int32, (_RB, _CB), 1)
    t = r * _CB + cc0
    s2 = jnp.where(t < l, s_scr[...], -jnp.inf)

    i = lax.bitcast_convert_type(s2, jnp.int32)
    skey = i ^ (lax.shift_right_arithmetic(i, 31) & jnp.int32(0x7FFFFFFF))

    lf = l.astype(jnp.float32) * jnp.float32(_FRAC)
    ki = lf.astype(jnp.int32)
    ki = ki + (ki.astype(jnp.float32) < lf).astype(jnp.int32)
    k = jnp.clip(ki, _MIN_K, T)

    p = jnp.int32(0)
    for shift, width in _ROUNDS:
        n = (1 << width) - 1
        oks = []
        for j in range(1, n + 1):
            cpat = p | (jnp.int32(j) << shift)
            cval = cpat ^ jnp.int32(_INT_MIN)
            cnt = jnp.sum((skey >= cval).astype(jnp.int32))
            oks.append((cnt >= k).astype(jnp.int32))
        j_star = oks[0]
        for o in oks[1:]:
            j_star = j_star + o
        p = p | (j_star << shift)
    theta = p ^ jnp.int32(_INT_MIN)

    cgt = jnp.sum((skey > theta).astype(jnp.int32))
    m = (k - cgt).astype(jnp.float32)

    eq = (skey == theta).astype(jnp.float32)
    cc = lax.broadcasted_iota(jnp.int32, (_CB, _CB), 0)
    cr = lax.broadcasted_iota(jnp.int32, (_CB, _CB), 1)
    lt_incl = (cc <= cr).astype(jnp.float32)
    lane_incl = jnp.dot(eq, lt_incl, preferred_element_type=jnp.float32)
    row_tot = jnp.sum(eq, axis=1, keepdims=True)
    ar = lax.broadcasted_iota(jnp.int32, (_RB, _RB), 0)
    ac = lax.broadcasted_iota(jnp.int32, (_RB, _RB), 1)
    strict = (ac < ar).astype(jnp.float32)
    row_excl = jnp.dot(strict, row_tot, preferred_element_type=jnp.float32)
    rank_excl = row_excl + lane_incl - eq

    w = jnp.where(
        (skey > theta) | ((skey == theta) & (rank_excl < m)),
        jnp.float32(1.0),
        jnp.float32(0.0),
    )

    wrow = w.reshape(1, T)
    pooled = (
        jnp.dot(wrow[:, : T // 2], xa_ref[0],
                preferred_element_type=jnp.float32)
        + jnp.dot(wrow[:, T // 2:], xb_ref[0],
                  preferred_element_type=jnp.float32)
    )
    out_ref[0] = pooled / k.astype(jnp.float32)


def kernel(x, lengths, W1, b1, W2):
    B, T, D = x.shape
    lengths = lengths.astype(jnp.int32)
    b1r = b1.reshape(1, _HIDDEN).astype(jnp.float32)
    xh = x.reshape(2 * B, T // 2, D)
    return pl.pallas_call(
        _row_kernel,
        grid=(B,),
        in_specs=[
            pl.BlockSpec(memory_space=pltpu.SMEM),
            pl.BlockSpec((1, T // 2, D), lambda b: (2 * b, 0, 0)),
            pl.BlockSpec((1, T // 2, D), lambda b: (2 * b + 1, 0, 0)),
            pl.BlockSpec((D, _HIDDEN), lambda b: (0, 0)),
            pl.BlockSpec((1, _HIDDEN), lambda b: (0, 0)),
            pl.BlockSpec((_HIDDEN, 1), lambda b: (0, 0)),
        ],
        out_specs=pl.BlockSpec((1, 1, D), lambda b: (b, 0, 0)),
        out_shape=jax.ShapeDtypeStruct((B, 1, D), jnp.float32),
        scratch_shapes=[pltpu.VMEM((_RB, _CB), jnp.float32)],
    )(lengths, xh, xh, W1, b1r, W2).reshape(B, D)
